# Initial kernel scaffold; baseline (speedup 1.0000x reference)
import jax
import jax.numpy as jnp
from jax.experimental import pallas as pl


def kernel(x_nchw, w_conv, gamma, beta):
    raise NotImplementedError("write your pallas kernel here")


# speedup vs baseline: 1.4564x; 1.4564x over previous
import functools

import jax
import jax.numpy as jnp
from jax import lax
from jax.experimental import pallas as pl
from jax.experimental.pallas import tpu as pltpu

_R = 2
_EPS = 1e-5


def _conv_stack(x_ref, m_ref, *, hw, w):
    xr = x_ref[0]
    slabs = []
    for t in range(9):
        ki, kj = divmod(t, 3)
        off = ki * w + kj
        slab = xr[:, off:off + hw]
        if kj == 0:
            slab = slab * m_ref[0:1, :]
        elif kj == 2:
            slab = slab * m_ref[1:2, :]
        slabs.append(slab)
    return jnp.concatenate(slabs, axis=0)


def _stats_kernel(x_ref, w_ref, m_ref, p_ref, *, hw, w):
    xs = _conv_stack(x_ref, m_ref, hw=hw, w=w)
    y = jnp.dot(w_ref[...], xs, preferred_element_type=jnp.float32)
    p_ref[0, :, 0:1] = jnp.sum(y, axis=1, keepdims=True)
    p_ref[0, :, 1:2] = jnp.sum(y * y, axis=1, keepdims=True)


def _finalize_kernel(p_ref, g_ref, b_ref, s_ref, *, inv_count, eps):
    s = jnp.sum(p_ref[...], axis=0)
    mean = s[:, 0:1] * inv_count
    var = s[:, 1:2] * inv_count - mean * mean
    scale = g_ref[...] * lax.rsqrt(var + eps)
    s_ref[:, 0:1] = scale
    s_ref[:, 1:2] = b_ref[...] - mean * scale


def _apply_kernel(x_ref, w_ref, m_ref, s_ref, o_ref, *, hw, w, c_out):
    xs = _conv_stack(x_ref, m_ref, hw=hw, w=w)
    y = jnp.dot(w_ref[...], xs, preferred_element_type=jnp.float32)
    z = y * s_ref[:, 0:1] + s_ref[:, 1:2]
    o_ref[0, :c_out, :] = z
    o_ref[0, c_out:, :] = z


def kernel(x_nchw, w_conv, gamma, beta):
    r, eps = _R, _EPS
    n, c_in, h, w = x_nchw.shape
    c_out = w_conv.shape[0]
    crr = c_out * r
    hw = h * w
    l = 1 + (h + 2) * w
    need = (2 * w + 2) + hw
    lpad = ((max(l, need) + 127) // 128) * 128

    xp = jnp.pad(x_nchw, ((0, 0), (0, 0), (1, 1), (0, 0))).reshape(n, c_in, (h + 2) * w)
    xd = jnp.pad(xp, ((0, 0), (0, 0), (1, lpad - 1 - (h + 2) * w))).astype(jnp.bfloat16)

    w_all = jnp.transpose(w_conv, (0, 2, 3, 1)).reshape(c_out, 9 * c_in)
    w_all = w_all.astype(jnp.bfloat16)

    col = jnp.arange(hw, dtype=jnp.int32) % w
    m = jnp.stack([(col != 0), (col != w - 1)]).astype(jnp.bfloat16)

    g2 = gamma.reshape(c_out, 1).astype(jnp.float32)
    b2 = beta.reshape(c_out, 1).astype(jnp.float32)

    x_spec = pl.BlockSpec((1, c_in, lpad), lambda i: (i, 0, 0))
    w_spec = pl.BlockSpec((c_out, 9 * c_in), lambda i: (0, 0))
    m_spec = pl.BlockSpec((2, hw), lambda i: (0, 0))

    partials = pl.pallas_call(
        functools.partial(_stats_kernel, hw=hw, w=w),
        grid=(n,),
        in_specs=[x_spec, w_spec, m_spec],
        out_specs=pl.BlockSpec((1, c_out, 2), lambda i: (i, 0, 0)),
        out_shape=jax.ShapeDtypeStruct((n, c_out, 2), jnp.float32),
        compiler_params=pltpu.CompilerParams(dimension_semantics=("parallel",)),
    )(xd, w_all, m)

    sb = pl.pallas_call(
        functools.partial(_finalize_kernel, inv_count=1.0 / float(n * h * w), eps=eps),
        out_shape=jax.ShapeDtypeStruct((c_out, 2), jnp.float32),
    )(partials, g2, b2)

    out = pl.pallas_call(
        functools.partial(_apply_kernel, hw=hw, w=w, c_out=c_out),
        grid=(n,),
        in_specs=[x_spec, w_spec, m_spec,
                  pl.BlockSpec((c_out, 2), lambda i: (0, 0))],
        out_specs=pl.BlockSpec((1, crr, hw), lambda i: (i, 0, 0)),
        out_shape=jax.ShapeDtypeStruct((n, crr, hw), jnp.float32),
        compiler_params=pltpu.CompilerParams(dimension_semantics=("parallel",)),
    )(xd, w_all, m, sb)

    return out.reshape(n, crr, h, w)


# speedup vs baseline: 1.7889x; 1.2283x over previous
import functools

import jax
import jax.numpy as jnp
from jax import lax
from jax.experimental import pallas as pl
from jax.experimental.pallas import tpu as pltpu

_R = 2
_EPS = 1e-5


def _conv_stack(x_ref, m_ref, *, hw, w, b):
    xr = x_ref[b]
    slabs = []
    for t in range(9):
        ki, kj = divmod(t, 3)
        off = ki * w + kj
        slab = xr[:, off:off + hw]
        if kj == 0:
            slab = slab * m_ref[0:1, :]
        elif kj == 2:
            slab = slab * m_ref[1:2, :]
        slabs.append(slab)
    return jnp.concatenate(slabs, axis=0)


def _stats_kernel(x_ref, w_ref, m_ref, p_ref, *, hw, w, ib):
    for b in range(ib):
        xs = _conv_stack(x_ref, m_ref, hw=hw, w=w, b=b)
        y = jnp.dot(w_ref[...], xs, preferred_element_type=jnp.float32)
        p_ref[b, :, 0:1] = jnp.sum(y, axis=1, keepdims=True)
        p_ref[b, :, 1:2] = jnp.sum(y * y, axis=1, keepdims=True)


def _finalize_kernel(p_ref, g_ref, b_ref, s_ref, *, inv_count, eps):
    s = jnp.sum(p_ref[...], axis=0)
    mean = s[:, 0:1] * inv_count
    var = s[:, 1:2] * inv_count - mean * mean
    scale = g_ref[...] * lax.rsqrt(var + eps)
    s_ref[:, 0:1] = scale
    s_ref[:, 1:2] = b_ref[...] - mean * scale


def _apply_kernel(x_ref, w_ref, m_ref, s_ref, o_ref, *, hw, w, c_out, ib):
    for b in range(ib):
        xs = _conv_stack(x_ref, m_ref, hw=hw, w=w, b=b)
        y = jnp.dot(w_ref[...], xs, preferred_element_type=jnp.float32)
        z = y * s_ref[:, 0:1] + s_ref[:, 1:2]
        o_ref[b, :c_out, :] = z
        o_ref[b, c_out:, :] = z


def kernel(x_nchw, w_conv, gamma, beta):
    r, eps = _R, _EPS
    n, c_in, h, w = x_nchw.shape
    c_out = w_conv.shape[0]
    crr = c_out * r
    hw = h * w
    l = 1 + (h + 2) * w
    need = (2 * w + 2) + hw
    lpad = ((max(l, need) + 127) // 128) * 128

    xp = jnp.pad(x_nchw, ((0, 0), (0, 0), (1, 1), (0, 0))).reshape(n, c_in, (h + 2) * w)
    xd = jnp.pad(xp, ((0, 0), (0, 0), (1, lpad - 1 - (h + 2) * w))).astype(jnp.bfloat16)

    w_all = jnp.transpose(w_conv, (0, 2, 3, 1)).reshape(c_out, 9 * c_in)
    w_all = w_all.astype(jnp.bfloat16)

    col = jnp.arange(hw, dtype=jnp.int32) % w
    m = jnp.stack([(col != 0), (col != w - 1)]).astype(jnp.bfloat16)

    g2 = gamma.reshape(c_out, 1).astype(jnp.float32)
    b2 = beta.reshape(c_out, 1).astype(jnp.float32)

    ib = 8
    while n % ib:
        ib //= 2

    x_spec = pl.BlockSpec((ib, c_in, lpad), lambda i: (i, 0, 0))
    w_spec = pl.BlockSpec((c_out, 9 * c_in), lambda i: (0, 0))
    m_spec = pl.BlockSpec((2, hw), lambda i: (0, 0))

    partials = pl.pallas_call(
        functools.partial(_stats_kernel, hw=hw, w=w, ib=ib),
        grid=(n // ib,),
        in_specs=[x_spec, w_spec, m_spec],
        out_specs=pl.BlockSpec((ib, c_out, 2), lambda i: (i, 0, 0)),
        out_shape=jax.ShapeDtypeStruct((n, c_out, 2), jnp.float32),
        compiler_params=pltpu.CompilerParams(dimension_semantics=("parallel",)),
    )(xd, w_all, m)

    sb = pl.pallas_call(
        functools.partial(_finalize_kernel, inv_count=1.0 / float(n * h * w), eps=eps),
        out_shape=jax.ShapeDtypeStruct((c_out, 2), jnp.float32),
    )(partials, g2, b2)

    out = pl.pallas_call(
        functools.partial(_apply_kernel, hw=hw, w=w, c_out=c_out, ib=ib),
        grid=(n // ib,),
        in_specs=[x_spec, w_spec, m_spec,
                  pl.BlockSpec((c_out, 2), lambda i: (0, 0))],
        out_specs=pl.BlockSpec((ib, crr, hw), lambda i: (i, 0, 0)),
        out_shape=jax.ShapeDtypeStruct((n, crr, hw), jnp.float32),
        compiler_params=pltpu.CompilerParams(dimension_semantics=("parallel",)),
    )(xd, w_all, m, sb)

    return out.reshape(n, crr, h, w)


# speedup vs baseline: 2.4668x; 1.3790x over previous
import functools

import jax
import jax.numpy as jnp
from jax import lax
from jax.experimental import pallas as pl
from jax.experimental.pallas import tpu as pltpu

_R = 2
_EPS = 1e-5


def _rolled(xb, rr, hw):
    if rr == 0:
        return xb
    return jnp.concatenate([xb[:, rr:], xb[:, :rr]], axis=1)


def _conv_stack(xb, m_ref, *, hw, w):
    slabs = []
    for t in range(9):
        ki, kj = divmod(t, 3)
        s = (ki - 1) * w + (kj - 1)
        slab = _rolled(xb, s % hw, hw)
        if t != 4:
            slab = slab * m_ref[t:t + 1, :]
        slabs.append(slab)
    return jnp.concatenate(slabs, axis=0)


def _stats_kernel(x_ref, w_ref, m_ref, p_ref, y_ref, *, hw, w, ib):
    for b in range(ib):
        xb = x_ref[b].astype(jnp.bfloat16)
        xs = _conv_stack(xb, m_ref, hw=hw, w=w)
        y = jnp.dot(w_ref[...], xs, preferred_element_type=jnp.float32)
        y_ref[b] = y.astype(jnp.bfloat16)
        p_ref[b, :, 0:1] = jnp.sum(y, axis=1, keepdims=True)
        p_ref[b, :, 1:2] = jnp.sum(y * y, axis=1, keepdims=True)


def _finalize_kernel(p_ref, g_ref, b_ref, s_ref, *, inv_count, eps):
    s = jnp.sum(p_ref[...], axis=0)
    mean = s[:, 0:1] * inv_count
    var = s[:, 1:2] * inv_count - mean * mean
    scale = g_ref[...] * lax.rsqrt(var + eps)
    s_ref[:, 0:1] = scale
    s_ref[:, 1:2] = b_ref[...] - mean * scale


def _apply_kernel(y_ref, s_ref, o_ref, *, c_out, ib):
    for b in range(ib):
        y = y_ref[b].astype(jnp.float32)
        z = y * s_ref[:, 0:1] + s_ref[:, 1:2]
        o_ref[b, :c_out, :] = z
        o_ref[b, c_out:, :] = z


def kernel(x_nchw, w_conv, gamma, beta):
    r, eps = _R, _EPS
    n, c_in, h, w = x_nchw.shape
    c_out = w_conv.shape[0]
    crr = c_out * r
    hw = h * w

    xflat = x_nchw.reshape(n, c_in, hw)

    w_all = jnp.transpose(w_conv, (0, 2, 3, 1)).reshape(c_out, 9 * c_in)
    w_all = w_all.astype(jnp.bfloat16)

    rows = jnp.arange(hw, dtype=jnp.int32) // w
    cols = jnp.arange(hw, dtype=jnp.int32) % w
    mk = []
    for t in range(9):
        ki, kj = divmod(t, 3)
        ri, cj = rows + (ki - 1), cols + (kj - 1)
        mk.append((ri >= 0) & (ri < h) & (cj >= 0) & (cj < w))
    m = jnp.concatenate(
        [jnp.stack(mk), jnp.ones((7, hw), dtype=jnp.bool_)]).astype(jnp.bfloat16)

    g2 = gamma.reshape(c_out, 1).astype(jnp.float32)
    b2 = beta.reshape(c_out, 1).astype(jnp.float32)

    ib = 8
    while n % ib:
        ib //= 2

    x_spec = pl.BlockSpec((ib, c_in, hw), lambda i: (i, 0, 0))
    w_spec = pl.BlockSpec((c_out, 9 * c_in), lambda i: (0, 0))
    m_spec = pl.BlockSpec((16, hw), lambda i: (0, 0))

    partials, yflat = pl.pallas_call(
        functools.partial(_stats_kernel, hw=hw, w=w, ib=ib),
        grid=(n // ib,),
        in_specs=[x_spec, w_spec, m_spec],
        out_specs=[
            pl.BlockSpec((ib, c_out, 2), lambda i: (i, 0, 0)),
            pl.BlockSpec((ib, c_out, hw), lambda i: (i, 0, 0)),
        ],
        out_shape=[
            jax.ShapeDtypeStruct((n, c_out, 2), jnp.float32),
            jax.ShapeDtypeStruct((n, c_out, hw), jnp.bfloat16),
        ],
        compiler_params=pltpu.CompilerParams(dimension_semantics=("parallel",)),
    )(xflat, w_all, m)

    sb = pl.pallas_call(
        functools.partial(_finalize_kernel, inv_count=1.0 / float(n * h * w), eps=eps),
        out_shape=jax.ShapeDtypeStruct((c_out, 2), jnp.float32),
    )(partials, g2, b2)

    out = pl.pallas_call(
        functools.partial(_apply_kernel, c_out=c_out, ib=ib),
        grid=(n // ib,),
        in_specs=[pl.BlockSpec((ib, c_out, hw), lambda i: (i, 0, 0)),
                  pl.BlockSpec((c_out, 2), lambda i: (0, 0))],
        out_specs=pl.BlockSpec((ib, crr, hw), lambda i: (i, 0, 0)),
        out_shape=jax.ShapeDtypeStruct((n, crr, hw), jnp.float32),
        compiler_params=pltpu.CompilerParams(dimension_semantics=("parallel",)),
    )(yflat, sb)

    return out.reshape(n, crr, h, w)
